# baseline (speedup 1.0000x reference)
import functools

import jax
import jax.numpy as jnp
from jax import lax
from jax.experimental import pallas as pl
from jax.experimental.pallas import tpu as pltpu
from jax.experimental.pallas import tpu_sc as plsc

VOCAB = 100000
EMBED = 64
BATCH = 4096
HIST = 200

NC = 2
NS = 16
LANES = 16
NW = NC * NS
BPW = BATCH // NW
HALF = HIST // 2
NVEC = EMBED // LANES

CB = 2
NBUF = 3
NENT = BPW // CB

_mesh = plsc.VectorSubcoreMesh(core_axis_name="c", subcore_axis_name="s")


@functools.partial(
    pl.kernel,
    out_type=jax.ShapeDtypeStruct((BATCH, EMBED), jnp.float32),
    mesh=_mesh,
    compiler_params=pltpu.CompilerParams(use_tc_tiling_on_sc=False,
                                         needs_layout_passes=False),
    scratch_types=[
        pltpu.VMEM((NENT * CB * HIST,), jnp.int32),
        pltpu.VMEM((NBUF, CB * HIST, EMBED), jnp.bfloat16),
        pltpu.VMEM((BPW, EMBED), jnp.float32),
        pltpu.SemaphoreType.DMA,
        pltpu.SemaphoreType.DMA,
        pltpu.SemaphoreType.DMA,
    ],
)
def _embed_pool(x_hbm, table_hbm, dummy_hbm, out_hbm, idx_v, rows_v, out_v,
                sem0, sem1, sem2):
    sems = (sem0, sem1, sem2)
    wid = lax.axis_index("s") * NC + lax.axis_index("c")
    base = wid * BPW

    pltpu.sync_copy(x_hbm.at[pl.ds(wid * BPW * HIST, BPW * HIST)], idx_v)

    def issue(e, buf):
        pltpu.async_copy(table_hbm.at[idx_v.at[pl.ds(e * CB * HIST, CB * HIST)]],
                         rows_v.at[buf], sems[buf])

    def wait(buf):
        pltpu.make_async_copy(dummy_hbm, rows_v.at[buf], sems[buf]).wait()

    i2 = 2 * lax.iota(jnp.int32, LANES)

    def reduce(b, buf, c):
        U = 4

        def body(t, acc):
            acc = list(acc)
            for u in range(U):
                s = 4 * (u % 2)
                l = c * HIST + t * U + u
                for k in range(2):
                    ab = rows_v[buf, l, pl.ds(32 * k, 32)]
                    ev, od = plsc.unpack(ab, format=plsc.PackFormat.INTERLEAVED)
                    acc[s + 2 * k] = acc[s + 2 * k] + ev
                    acc[s + 2 * k + 1] = acc[s + 2 * k + 1] + od
            return tuple(acc)

        acc = lax.fori_loop(
            0, HIST // U, body,
            tuple(jnp.zeros((LANES,), jnp.float32) for _ in range(8)))
        rowv = jnp.full((LANES,), b, dtype=jnp.int32)
        for k in range(2):
            va = (acc[2 * k] + acc[4 + 2 * k]) * (1.0 / HIST)
            vb = (acc[2 * k + 1] + acc[4 + 2 * k + 1]) * (1.0 / HIST)
            plsc.store_scatter(out_v, [rowv, i2 + 32 * k], va)
            plsc.store_scatter(out_v, [rowv, i2 + 32 * k + 1], vb)

    for e in range(NBUF - 1):
        issue(e, e)

    n_groups = -(-NENT // NBUF)

    def outer(j, _):
        e0 = j * NBUF
        for u in range(NBUF):
            ne = e0 + u + NBUF - 1

            @pl.when(ne < NENT)
            def _():
                issue(ne, (u + NBUF - 1) % NBUF)

            @pl.when(e0 + u < NENT)
            def _():
                wait(u)
                for c in range(CB):
                    reduce((e0 + u) * CB + c, u, c)
        return 0

    lax.fori_loop(0, n_groups, outer, 0)
    pltpu.sync_copy(out_v, out_hbm.at[pl.ds(base, BPW)])


def kernel(x, table):
    x3 = x.astype(jnp.int32).reshape(BATCH * HIST)
    dummy = jnp.zeros((CB * HIST, EMBED), jnp.bfloat16)
    return _embed_pool(x3, table.astype(jnp.bfloat16), dummy)
